# baseline (speedup 1.0000x reference)
import functools

import jax
import jax.numpy as jnp
from jax import lax
from jax.experimental import pallas as pl
from jax.experimental.pallas import tpu as pltpu
from jax.experimental.pallas import tpu_sc as plsc

B = 1024
IN_DIM = 768
T = 4
E_DIM = 256
K = 8192
N_LAYERS = 4
BETA = 0.65
ALIGN_DIM = 512
R = B * T

NC = 2
NS = 16
NW = NC * NS
RPW = R // NW

RB = 256
KB = 1024
RT = R // RB
KT = K // KB


def _enc_body(x_ref, w_ref, b_ref, o_ref):
    o_ref[...] = (
        jnp.dot(x_ref[...], w_ref[...], preferred_element_type=jnp.float32)
        + b_ref[...]
    )


def _encode(x, w, b):
    return pl.pallas_call(
        _enc_body,
        out_shape=jax.ShapeDtypeStruct((B, T * E_DIM), jnp.float32),
    )(x, w, b.reshape(1, T * E_DIM))


def _dist_body(res_ref, cb_ref, idx_ref, c2_ref, min_ref, arg_ref):
    kt = pl.program_id(0)
    rt = pl.program_id(1)

    @pl.when(rt == 0)
    def _():
        cb = cb_ref[...]
        c2_ref[...] = jnp.sum(cb * cb, axis=1)[None, :]

    res = res_ref[...]
    r2 = jnp.sum(res * res, axis=1, keepdims=True)
    dots = lax.dot_general(
        res, cb_ref[...], (((1,), (1,)), ((), ())),
        preferred_element_type=jnp.float32,
    )
    dist = (r2 - 2.0 * dots) + c2_ref[...]
    bmin = jnp.min(dist, axis=1, keepdims=True)
    ii = lax.broadcasted_iota(jnp.int32, dist.shape, 1) + kt * KB
    bidx = jnp.min(
        jnp.where(dist == bmin, ii, jnp.int32(2**30)), axis=1, keepdims=True
    )
    sl = pl.ds(rt * RB, RB)

    @pl.when(kt == 0)
    def _():
        min_ref[sl, :] = bmin
        arg_ref[sl, :] = bidx

    @pl.when(kt > 0)
    def _():
        om = min_ref[sl, :]
        oi = arg_ref[sl, :]
        better = bmin < om
        min_ref[sl, :] = jnp.where(better, bmin, om)
        arg_ref[sl, :] = jnp.where(better, bidx, oi)

    idx_ref[0, 0, 0, :] = arg_ref[sl, :][:, 0]


def _nearest(res, cb):
    idx3 = pl.pallas_call(
        _dist_body,
        grid=(KT, RT),
        in_specs=[
            pl.BlockSpec((RB, E_DIM), lambda kt, rt: (rt, 0)),
            pl.BlockSpec((KB, E_DIM), lambda kt, rt: (kt, 0)),
        ],
        out_specs=pl.BlockSpec((1, 1, 1, RB), lambda kt, rt: (kt, rt, 0, 0)),
        out_shape=jax.ShapeDtypeStruct((KT, RT, 1, RB), jnp.int32),
        scratch_shapes=[
            pltpu.VMEM((1, KB), jnp.float32),
            pltpu.VMEM((R, 1), jnp.float32),
            pltpu.VMEM((R, 1), jnp.int32),
        ],
    )(res, cb)
    return idx3[KT - 1].reshape(R)


def _sc_update_body(cb_hbm, idx_hbm, res_hbm, res_out, sq_out,
                    idx_v, q_v, r_v, sq_v, sem):
    wid = lax.axis_index("s") * NC + lax.axis_index("c")
    base = wid * RPW
    pltpu.sync_copy(idx_hbm.at[pl.ds(base, RPW)], idx_v)
    pltpu.async_copy(cb_hbm.at[idx_v], q_v, sem).wait()
    pltpu.sync_copy(res_hbm.at[pl.ds(base, RPW)], r_v)

    def row(i, acc):
        for j in range(E_DIM // 16):
            sl = pl.ds(j * 16, 16)
            r = r_v[i, sl]
            q = q_v[i, sl]
            t = q - r
            q_st = r + t
            r_v[i, sl] = r - q_st
            acc = acc + t * t
        return acc

    acc = lax.fori_loop(0, RPW, row, jnp.zeros((16,), jnp.float32))
    sq_v[...] = acc
    pltpu.sync_copy(r_v, res_out.at[pl.ds(base, RPW)])
    pltpu.sync_copy(sq_v, sq_out.at[wid])


@functools.lru_cache(maxsize=1)
def _build_sc_update():
    return pl.kernel(
        _sc_update_body,
        out_type=(
            jax.ShapeDtypeStruct((R, E_DIM), jnp.float32),
            jax.ShapeDtypeStruct((NW, 16), jnp.float32),
        ),
        mesh=plsc.VectorSubcoreMesh(core_axis_name="c", subcore_axis_name="s",
                                    num_cores=NC, num_subcores=NS),
        scratch_types=[
            pltpu.VMEM((RPW,), jnp.int32),
            pltpu.VMEM((RPW, E_DIM), jnp.float32),
            pltpu.VMEM((RPW, E_DIM), jnp.float32),
            pltpu.VMEM((16,), jnp.float32),
            pltpu.SemaphoreType.DMA,
        ],
    )


DB = 256
DT = B // DB


def _dec_body(res_ref, xe_ref, wdt_ref, bdt_ref, wd_ref, bd_ref,
              wa_ref, ba_ref, sq_ref, xd_ref, rec_ref, al_ref, loss_ref):
    t = pl.program_id(0)
    q = xe_ref[...] - res_ref[...]
    wdt = wdt_ref[...]
    bdt = bdt_ref[...]
    xd_parts = []
    al_parts = []
    for tok in range(T):
        qt = q[:, tok * E_DIM:(tok + 1) * E_DIM]
        xdt = jnp.dot(qt, wdt, preferred_element_type=jnp.float32) + bdt
        xd_parts.append(xdt)
        al_parts.append(
            jnp.dot(xdt, wa_ref[...], preferred_element_type=jnp.float32)
            + ba_ref[...]
        )
    xd = jnp.concatenate(xd_parts, axis=1)
    xd_ref[...] = xd
    rec_ref[...] = (
        jnp.dot(xd, wd_ref[...], preferred_element_type=jnp.float32)
        + bd_ref[...]
    )
    al_ref[...] = jnp.concatenate(al_parts, axis=1)

    @pl.when(t == 0)
    def _():
        loss_ref[0, 0] = (
            (1.0 + BETA) * jnp.sum(sq_ref[...]) / jnp.float32(R * E_DIM)
        )


def _decode(res4, xe, wdt, bdt, wd, bd, wa, ba, sq):
    return pl.pallas_call(
        _dec_body,
        grid=(DT,),
        in_specs=[
            pl.BlockSpec((DB, T * E_DIM), lambda t: (t, 0)),
            pl.BlockSpec((DB, T * E_DIM), lambda t: (t, 0)),
            pl.BlockSpec((E_DIM, E_DIM), lambda t: (0, 0)),
            pl.BlockSpec((1, E_DIM), lambda t: (0, 0)),
            pl.BlockSpec((T * E_DIM, IN_DIM), lambda t: (0, 0)),
            pl.BlockSpec((1, IN_DIM), lambda t: (0, 0)),
            pl.BlockSpec((E_DIM, ALIGN_DIM), lambda t: (0, 0)),
            pl.BlockSpec((1, ALIGN_DIM), lambda t: (0, 0)),
            pl.BlockSpec((N_LAYERS * NW, 16), lambda t: (0, 0)),
        ],
        out_specs=[
            pl.BlockSpec((DB, T * E_DIM), lambda t: (t, 0)),
            pl.BlockSpec((DB, IN_DIM), lambda t: (t, 0)),
            pl.BlockSpec((DB, T * ALIGN_DIM), lambda t: (t, 0)),
            pl.BlockSpec(memory_space=pltpu.SMEM),
        ],
        out_shape=[
            jax.ShapeDtypeStruct((B, T * E_DIM), jnp.float32),
            jax.ShapeDtypeStruct((B, IN_DIM), jnp.float32),
            jax.ShapeDtypeStruct((B, T * ALIGN_DIM), jnp.float32),
            jax.ShapeDtypeStruct((1, 1), jnp.float32),
        ],
    )(res4, xe, wdt, bdt.reshape(1, E_DIM), wd, bd.reshape(1, IN_DIM),
      wa, ba.reshape(1, ALIGN_DIM), sq)


def kernel(video_patches, W_enc, b_enc, cb0, cb1, cb2, cb3,
           W_dec_tok, b_dec_tok, W_dec, b_dec, W_align, b_align):
    xe = _encode(video_patches, W_enc, b_enc)
    res = xe.reshape(R, E_DIM)
    idxs = []
    sqs = []
    for cb in (cb0, cb1, cb2, cb3):
        idx = _nearest(res, cb)
        res, sq = _build_sc_update()(cb, idx, res)
        idxs.append(idx)
        sqs.append(sq)
    sq_all = jnp.concatenate(sqs, axis=0)
    xd, rec, al, loss = _decode(
        res.reshape(B, T * E_DIM), xe, W_dec_tok, b_dec_tok,
        W_dec, b_dec, W_align, b_align, sq_all)
    indices = jnp.stack(idxs, axis=-1).reshape(B, T, N_LAYERS)
    return (
        rec,
        loss.reshape(()),
        indices,
        xe.reshape(B, T, E_DIM),
        xd.reshape(B, T, E_DIM),
        al.reshape(B, T, ALIGN_DIM),
    )


# speedup vs baseline: 1.1785x; 1.1785x over previous
import functools

import jax
import jax.numpy as jnp
from jax import lax
from jax.experimental import pallas as pl
from jax.experimental.pallas import tpu as pltpu
from jax.experimental.pallas import tpu_sc as plsc

B = 1024
IN_DIM = 768
T = 4
E_DIM = 256
K = 8192
N_LAYERS = 4
BETA = 0.65
ALIGN_DIM = 512
R = B * T

NC = 2
NS = 16
NW = NC * NS
RPW = R // NW

RB = 256
KB = 1024
RT = R // RB
KT = K // KB


def _enc_body(x_ref, w_ref, b_ref, o_ref):
    o_ref[...] = (
        jnp.dot(x_ref[...], w_ref[...], preferred_element_type=jnp.float32)
        + b_ref[...]
    )


def _encode(x, w, b):
    return pl.pallas_call(
        _enc_body,
        out_shape=jax.ShapeDtypeStruct((B, T * E_DIM), jnp.float32),
    )(x, w, b.reshape(1, T * E_DIM))


def _dist_body(res_ref, cb_ref, idx_ref, c2_ref, r2_ref, min_ref, cid_ref):
    kt = pl.program_id(0)
    rt = pl.program_id(1)
    sl = pl.ds(rt * RB, RB)
    ones = jnp.ones((1, E_DIM), jnp.float32)

    @pl.when(rt == 0)
    def _():
        cb = cb_ref[...]
        c2_ref[...] = lax.dot_general(
            cb * cb, ones, (((1,), (1,)), ((), ())),
            preferred_element_type=jnp.float32,
        )

    res = res_ref[...]

    @pl.when(kt == 0)
    def _():
        r2_ref[:, sl] = lax.dot_general(
            ones, res * res, (((1,), (1,)), ((), ())),
            preferred_element_type=jnp.float32,
        )

    dots = lax.dot_general(
        cb_ref[...], res, (((1,), (1,)), ((), ())),
        preferred_element_type=jnp.float32,
    )
    dist = (r2_ref[:, sl] - 2.0 * dots) + c2_ref[...]
    dist3 = dist.reshape(KB // 8, 8, RB)
    bmin = jnp.min(dist3, axis=0)
    cc = lax.broadcasted_iota(jnp.int32, dist3.shape, 0)
    bcid = jnp.min(jnp.where(dist3 == bmin[None], cc, jnp.int32(2**30)), axis=0)

    @pl.when(kt == 0)
    def _():
        min_ref[:, sl] = bmin
        cid_ref[:, sl] = bcid

    @pl.when(kt > 0)
    def _():
        om = min_ref[:, sl]
        oi = cid_ref[:, sl]
        better = bmin < om
        min_ref[:, sl] = jnp.where(better, bmin, om)
        cid_ref[:, sl] = jnp.where(better, bcid + kt * (KB // 8), oi)

    mv = min_ref[:, sl]
    ki = cid_ref[:, sl] * 8 + lax.broadcasted_iota(jnp.int32, (8, RB), 0)
    m = jnp.min(mv, axis=0, keepdims=True)
    idx_ref[0, 0, :, :] = jnp.min(
        jnp.where(mv == m, ki, jnp.int32(2**30)), axis=0, keepdims=True
    )


def _nearest(res, cb):
    idx3 = pl.pallas_call(
        _dist_body,
        grid=(KT, RT),
        in_specs=[
            pl.BlockSpec((RB, E_DIM), lambda kt, rt: (rt, 0)),
            pl.BlockSpec((KB, E_DIM), lambda kt, rt: (kt, 0)),
        ],
        out_specs=pl.BlockSpec((1, 1, 1, RB), lambda kt, rt: (kt, rt, 0, 0)),
        out_shape=jax.ShapeDtypeStruct((KT, RT, 1, RB), jnp.int32),
        scratch_shapes=[
            pltpu.VMEM((KB, 1), jnp.float32),
            pltpu.VMEM((1, R), jnp.float32),
            pltpu.VMEM((8, R), jnp.float32),
            pltpu.VMEM((8, R), jnp.int32),
        ],
    )(res, cb)
    return idx3[KT - 1].reshape(R)


def _sc_update_body(cb_hbm, idx_hbm, res_hbm, res_out, sq_out,
                    idx_v, q_v, r_v, sq_v, sem):
    wid = lax.axis_index("s") * NC + lax.axis_index("c")
    base = wid * RPW
    pltpu.sync_copy(idx_hbm.at[pl.ds(base, RPW)], idx_v)
    pltpu.async_copy(cb_hbm.at[idx_v], q_v, sem).wait()
    pltpu.sync_copy(res_hbm.at[pl.ds(base, RPW)], r_v)

    def row(i, acc):
        for j in range(E_DIM // 16):
            sl = pl.ds(j * 16, 16)
            r = r_v[i, sl]
            q = q_v[i, sl]
            t = q - r
            q_st = r + t
            r_v[i, sl] = r - q_st
            acc = acc + t * t
        return acc

    acc = lax.fori_loop(0, RPW, row, jnp.zeros((16,), jnp.float32))
    sq_v[...] = acc
    pltpu.sync_copy(r_v, res_out.at[pl.ds(base, RPW)])
    pltpu.sync_copy(sq_v, sq_out.at[wid])


@functools.lru_cache(maxsize=1)
def _build_sc_update():
    return pl.kernel(
        _sc_update_body,
        out_type=(
            jax.ShapeDtypeStruct((R, E_DIM), jnp.float32),
            jax.ShapeDtypeStruct((NW, 16), jnp.float32),
        ),
        mesh=plsc.VectorSubcoreMesh(core_axis_name="c", subcore_axis_name="s",
                                    num_cores=NC, num_subcores=NS),
        scratch_types=[
            pltpu.VMEM((RPW,), jnp.int32),
            pltpu.VMEM((RPW, E_DIM), jnp.float32),
            pltpu.VMEM((RPW, E_DIM), jnp.float32),
            pltpu.VMEM((16,), jnp.float32),
            pltpu.SemaphoreType.DMA,
        ],
    )


DB = 256
DT = B // DB


def _dec_body(res_ref, xe_ref, wdt_ref, bdt_ref, wd_ref, bd_ref,
              wa_ref, ba_ref, sq_ref, xd_ref, rec_ref, al_ref, loss_ref):
    t = pl.program_id(0)
    q = xe_ref[...] - res_ref[...]
    wdt = wdt_ref[...]
    bdt = bdt_ref[...]
    xd_parts = []
    al_parts = []
    for tok in range(T):
        qt = q[:, tok * E_DIM:(tok + 1) * E_DIM]
        xdt = jnp.dot(qt, wdt, preferred_element_type=jnp.float32) + bdt
        xd_parts.append(xdt)
        al_parts.append(
            jnp.dot(xdt, wa_ref[...], preferred_element_type=jnp.float32)
            + ba_ref[...]
        )
    xd = jnp.concatenate(xd_parts, axis=1)
    xd_ref[...] = xd
    rec_ref[...] = (
        jnp.dot(xd, wd_ref[...], preferred_element_type=jnp.float32)
        + bd_ref[...]
    )
    al_ref[...] = jnp.concatenate(al_parts, axis=1)

    @pl.when(t == 0)
    def _():
        loss_ref[0, 0] = (
            (1.0 + BETA) * jnp.sum(sq_ref[...]) / jnp.float32(R * E_DIM)
        )


def _decode(res4, xe, wdt, bdt, wd, bd, wa, ba, sq):
    return pl.pallas_call(
        _dec_body,
        grid=(DT,),
        in_specs=[
            pl.BlockSpec((DB, T * E_DIM), lambda t: (t, 0)),
            pl.BlockSpec((DB, T * E_DIM), lambda t: (t, 0)),
            pl.BlockSpec((E_DIM, E_DIM), lambda t: (0, 0)),
            pl.BlockSpec((1, E_DIM), lambda t: (0, 0)),
            pl.BlockSpec((T * E_DIM, IN_DIM), lambda t: (0, 0)),
            pl.BlockSpec((1, IN_DIM), lambda t: (0, 0)),
            pl.BlockSpec((E_DIM, ALIGN_DIM), lambda t: (0, 0)),
            pl.BlockSpec((1, ALIGN_DIM), lambda t: (0, 0)),
            pl.BlockSpec((N_LAYERS * NW, 16), lambda t: (0, 0)),
        ],
        out_specs=[
            pl.BlockSpec((DB, T * E_DIM), lambda t: (t, 0)),
            pl.BlockSpec((DB, IN_DIM), lambda t: (t, 0)),
            pl.BlockSpec((DB, T * ALIGN_DIM), lambda t: (t, 0)),
            pl.BlockSpec(memory_space=pltpu.SMEM),
        ],
        out_shape=[
            jax.ShapeDtypeStruct((B, T * E_DIM), jnp.float32),
            jax.ShapeDtypeStruct((B, IN_DIM), jnp.float32),
            jax.ShapeDtypeStruct((B, T * ALIGN_DIM), jnp.float32),
            jax.ShapeDtypeStruct((1, 1), jnp.float32),
        ],
    )(res4, xe, wdt, bdt.reshape(1, E_DIM), wd, bd.reshape(1, IN_DIM),
      wa, ba.reshape(1, ALIGN_DIM), sq)


def kernel(video_patches, W_enc, b_enc, cb0, cb1, cb2, cb3,
           W_dec_tok, b_dec_tok, W_dec, b_dec, W_align, b_align):
    xe = _encode(video_patches, W_enc, b_enc)
    res = xe.reshape(R, E_DIM)
    idxs = []
    sqs = []
    for cb in (cb0, cb1, cb2, cb3):
        idx = _nearest(res, cb)
        res, sq = _build_sc_update()(cb, idx, res)
        idxs.append(idx)
        sqs.append(sq)
    sq_all = jnp.concatenate(sqs, axis=0)
    xd, rec, al, loss = _decode(
        res.reshape(B, T * E_DIM), xe, W_dec_tok, b_dec_tok,
        W_dec, b_dec, W_align, b_align, sq_all)
    indices = jnp.stack(idxs, axis=-1).reshape(B, T, N_LAYERS)
    return (
        rec,
        loss.reshape(()),
        indices,
        xe.reshape(B, T, E_DIM),
        xd.reshape(B, T, E_DIM),
        al.reshape(B, T, ALIGN_DIM),
    )
